# baseline (speedup 1.0000x reference)
import functools

import jax
import jax.numpy as jnp
from jax import lax
from jax.experimental import pallas as pl
from jax.experimental.pallas import tpu as pltpu
from jax.experimental.pallas import tpu_sc as plsc

N = 10000
D = 128
E = 320000

NC = 2
NS = 16
NW = NC * NS
CHUNK = 128
N_PAD = 10240
ROWS_PER_TILE = N_PAD // NS
E_PER_TILE_CHUNKS = 80
E_PER_TILE = E_PER_TILE_CHUNKS * CHUNK
E_PAD = NW * E_PER_TILE
PART = 40
CHUNKS_C0 = 40
CHUNKS_C1 = 120


def _sc_mesh():
    return plsc.VectorSubcoreMesh(
        core_axis_name="c", subcore_axis_name="s", num_cores=NC,
        num_subcores=NS)


def _sc_hist_body(dst_hbm, deg_out, idx_v, ones_v, zero_v, deg_sh):
    c = lax.axis_index("c")
    s = lax.axis_index("s")
    wid = c * NS + s
    zeros16 = jnp.zeros((16,), jnp.float32)
    ones16 = jnp.ones((16,), jnp.float32)
    for i in range(ROWS_PER_TILE // 16):
        zero_v[pl.ds(i * 16, 16)] = zeros16
    for i in range(CHUNK // 16):
        ones_v[pl.ds(i * 16, 16)] = ones16
    pltpu.sync_copy(zero_v, deg_sh.at[pl.ds(s * ROWS_PER_TILE, ROWS_PER_TILE)])
    plsc.subcore_barrier()
    pltpu.sync_copy(dst_hbm.at[wid], idx_v)

    def body(j, carry):
        pltpu.sync_copy(ones_v, deg_sh.at[idx_v.at[j]], add=True)
        return carry

    lax.fori_loop(0, E_PER_TILE_CHUNKS, body, 0)
    plsc.subcore_barrier()
    pltpu.sync_copy(deg_sh.at[pl.ds(s * ROWS_PER_TILE, ROWS_PER_TILE)],
                    deg_out.at[c, pl.ds(s * ROWS_PER_TILE, ROWS_PER_TILE)])


_sc_hist = pl.kernel(
    _sc_hist_body,
    out_type=jax.ShapeDtypeStruct((NC, N_PAD), jnp.float32),
    mesh=_sc_mesh(),
    scratch_types=[
        pltpu.VMEM((E_PER_TILE_CHUNKS, CHUNK), jnp.int32),
        pltpu.VMEM((CHUNK,), jnp.float32),
        pltpu.VMEM((ROWS_PER_TILE,), jnp.float32),
        pltpu.VMEM_SHARED((N_PAD,), jnp.float32),
    ],
)


def _sc_agg_body(srcA_hbm, dstA_hbm, srcB_hbm, dstB_hbm, g_hbm, acc_out,
                 isrc_v, idst_v, rows0, rows1, acc_sh, gsem0, gsem1):
    rows = [rows0, rows1]
    gsem = [gsem0, gsem1]
    c = lax.axis_index("c")
    s = lax.axis_index("s")
    zeros16 = jnp.zeros((16,), jnp.float32)

    def zbody(j, carry):
        for k in range(D // 16):
            rows0[j, pl.ds(k * 16, 16)] = zeros16
        return carry

    lax.fori_loop(0, CHUNK, zbody, 0)
    for k in range(ROWS_PER_TILE // CHUNK):
        pltpu.sync_copy(
            rows0, acc_sh.at[pl.ds(s * ROWS_PER_TILE + k * CHUNK, CHUNK)])
    plsc.subcore_barrier()

    def run(src_ref, dst_ref, nparts):
        for part in range(nparts):
            pltpu.sync_copy(src_ref.at[s, pl.ds(part * PART, PART)], isrc_v)
            pltpu.sync_copy(dst_ref.at[s, pl.ds(part * PART, PART)], idst_v)
            pltpu.async_copy(g_hbm.at[isrc_v.at[0]], rows0, gsem[0])

            def body(jj, carry):
                for t in range(2):
                    j = jj * 2 + t
                    b, bn = rows[t], rows[1 - t]
                    sb, sbn = gsem[t], gsem[1 - t]
                    pltpu.make_async_copy(g_hbm.at[isrc_v.at[j]], b,
                                          sb).wait()

                    def prefetch(j=j, bn=bn, sbn=sbn):
                        pltpu.async_copy(g_hbm.at[isrc_v.at[j + 1]], bn, sbn)

                    pl.when(j < PART - 1)(prefetch)
                    pltpu.sync_copy(b, acc_sh.at[idst_v.at[j]], add=True)
                return carry

            lax.fori_loop(0, PART // 2, body, 0)

    pl.when(c == 0)(lambda: run(srcA_hbm, dstA_hbm, CHUNKS_C0 // PART))
    pl.when(c == 1)(lambda: run(srcB_hbm, dstB_hbm, CHUNKS_C1 // PART))
    plsc.subcore_barrier()
    pltpu.sync_copy(acc_sh.at[pl.ds(s * ROWS_PER_TILE, ROWS_PER_TILE)],
                    acc_out.at[c, pl.ds(s * ROWS_PER_TILE, ROWS_PER_TILE)])


_sc_agg = pl.kernel(
    _sc_agg_body,
    out_type=jax.ShapeDtypeStruct((NC, N_PAD, D), jnp.float32),
    mesh=_sc_mesh(),
    scratch_types=[
        pltpu.VMEM((PART, CHUNK), jnp.int32),
        pltpu.VMEM((PART, CHUNK), jnp.int32),
        pltpu.VMEM((CHUNK, D), jnp.float32),
        pltpu.VMEM((CHUNK, D), jnp.float32),
        pltpu.VMEM_SHARED((N_PAD, D), jnp.float32),
        pltpu.SemaphoreType.DMA,
        pltpu.SemaphoreType.DMA,
    ],
)


def _tc_transform_body(x_ref, w_ref, deg0_ref, deg1_ref, h_ref, g_ref):
    h = jnp.dot(x_ref[...], w_ref[...], preferred_element_type=jnp.float32)
    deg = deg0_ref[...] + deg1_ref[...] + 1.0
    dis = lax.rsqrt(deg)
    h_ref[...] = h
    g_ref[...] = h * dis


def _tc_transform(x, W, deg0, deg1):
    R = 1000
    grid = (N // R,)
    return pl.pallas_call(
        _tc_transform_body,
        grid=grid,
        in_specs=[
            pl.BlockSpec((R, D), lambda i: (i, 0)),
            pl.BlockSpec((D, D), lambda i: (0, 0)),
            pl.BlockSpec((R, 1), lambda i: (i, 0)),
            pl.BlockSpec((R, 1), lambda i: (i, 0)),
        ],
        out_specs=[
            pl.BlockSpec((R, D), lambda i: (i, 0)),
            pl.BlockSpec((R, D), lambda i: (i, 0)),
        ],
        out_shape=[
            jax.ShapeDtypeStruct((N, D), jnp.float32),
            jax.ShapeDtypeStruct((N, D), jnp.float32),
        ],
    )(x, W, deg0, deg1)


def _tc_final_body(h_ref, deg0_ref, deg1_ref, acc0_ref, acc1_ref, b_ref,
                   gamma_ref, beta_ref, a_ref, o_ref):
    deg = deg0_ref[...] + deg1_ref[...] + 1.0
    dis = lax.rsqrt(deg)
    acc = acc0_ref[...] + acc1_ref[...]
    out = acc * dis + h_ref[...] * (1.0 / deg) + b_ref[...]
    mu = jnp.mean(out, axis=1, keepdims=True)
    cen = out - mu
    var = jnp.mean(cen * cen, axis=1, keepdims=True)
    y = cen * lax.rsqrt(var + 1e-5) * gamma_ref[...] + beta_ref[...]
    o_ref[...] = jnp.where(y >= 0.0, y, a_ref[...] * y)


def _tc_final(h, deg0, deg1, acc0, acc1, b, gamma, beta, a):
    R = 1000
    grid = (N // R,)
    full = lambda i: (0, 0)
    return pl.pallas_call(
        _tc_final_body,
        grid=grid,
        in_specs=[
            pl.BlockSpec((R, D), lambda i: (i, 0)),
            pl.BlockSpec((R, 1), lambda i: (i, 0)),
            pl.BlockSpec((R, 1), lambda i: (i, 0)),
            pl.BlockSpec((R, D), lambda i: (i, 0)),
            pl.BlockSpec((R, D), lambda i: (i, 0)),
            pl.BlockSpec((1, D), full),
            pl.BlockSpec((1, D), full),
            pl.BlockSpec((1, D), full),
            pl.BlockSpec((1, 1), full),
        ],
        out_specs=pl.BlockSpec((R, D), lambda i: (i, 0)),
        out_shape=jax.ShapeDtypeStruct((N, D), jnp.float32),
    )(h, deg0, deg1, acc0, acc1, b, gamma, beta, a)


def kernel(x, edge_index, batch, W, b, gamma, beta, a):
    del batch
    src = edge_index[0].astype(jnp.int32)
    dst = edge_index[1].astype(jnp.int32)
    n_dummy = E_PAD - E
    src_p = jnp.concatenate([src, jnp.zeros((n_dummy,), jnp.int32)])
    dst_p = jnp.concatenate(
        [dst, N + (jnp.arange(n_dummy, dtype=jnp.int32) % (N_PAD - N))])
    dst_r = dst_p.reshape(NW, E_PER_TILE_CHUNKS, CHUNK)
    e_a = NS * CHUNKS_C0 * CHUNK
    src_a = src_p[:e_a].reshape(NS, CHUNKS_C0, CHUNK)
    dst_a = dst_p[:e_a].reshape(NS, CHUNKS_C0, CHUNK)
    src_b = src_p[e_a:].reshape(NS, CHUNKS_C1, CHUNK)
    dst_b = dst_p[e_a:].reshape(NS, CHUNKS_C1, CHUNK)

    degp = _sc_hist(dst_r)
    deg0 = degp[0, :N].reshape(N, 1)
    deg1 = degp[1, :N].reshape(N, 1)

    h, g = _tc_transform(x, W, deg0, deg1)

    acc = _sc_agg(src_a, dst_a, src_b, dst_b, g)
    acc0 = acc[0, :N]
    acc1 = acc[1, :N]

    b2 = b.reshape(1, D)
    gamma2 = gamma.reshape(1, D)
    beta2 = beta.reshape(1, D)
    a2 = a.reshape(1, 1)
    return _tc_final(h, deg0, deg1, acc0, acc1, b2, gamma2, beta2, a2)


# speedup vs baseline: 1.1848x; 1.1848x over previous
import functools

import jax
import jax.numpy as jnp
from jax import lax
from jax.experimental import pallas as pl
from jax.experimental.pallas import tpu as pltpu
from jax.experimental.pallas import tpu_sc as plsc

N = 10000
D = 128
E = 320000

NC = 2
NS = 16
NW = NC * NS
CHUNK = 128
N_PAD = 10240
ROWS_PER_TILE = N_PAD // NS
E_PER_TILE_CHUNKS = 80
E_PER_TILE = E_PER_TILE_CHUNKS * CHUNK
E_PAD = NW * E_PER_TILE
PART = 40
CHUNKS_C0 = 120
CHUNKS_C1 = 40


def _sc_mesh():
    return plsc.VectorSubcoreMesh(
        core_axis_name="c", subcore_axis_name="s", num_cores=NC,
        num_subcores=NS)


def _sc_hist_body(dst_hbm, deg_out, idx_v, ones_v, zero_v, deg_sh):
    c = lax.axis_index("c")
    s = lax.axis_index("s")
    wid = c * NS + s
    zeros16 = jnp.zeros((16,), jnp.float32)
    ones16 = jnp.ones((16,), jnp.float32)
    for i in range(ROWS_PER_TILE // 16):
        zero_v[pl.ds(i * 16, 16)] = zeros16
    for i in range(CHUNK // 16):
        ones_v[pl.ds(i * 16, 16)] = ones16
    pltpu.sync_copy(zero_v, deg_sh.at[pl.ds(s * ROWS_PER_TILE, ROWS_PER_TILE)])
    plsc.subcore_barrier()
    pltpu.sync_copy(dst_hbm.at[wid], idx_v)

    def body(j, carry):
        pltpu.sync_copy(ones_v, deg_sh.at[idx_v.at[j]], add=True)
        return carry

    lax.fori_loop(0, E_PER_TILE_CHUNKS, body, 0)
    plsc.subcore_barrier()
    pltpu.sync_copy(deg_sh.at[pl.ds(s * ROWS_PER_TILE, ROWS_PER_TILE)],
                    deg_out.at[c, pl.ds(s * ROWS_PER_TILE, ROWS_PER_TILE)])


_sc_hist = pl.kernel(
    _sc_hist_body,
    out_type=jax.ShapeDtypeStruct((NC, N_PAD), jnp.float32),
    mesh=_sc_mesh(),
    scratch_types=[
        pltpu.VMEM((E_PER_TILE_CHUNKS, CHUNK), jnp.int32),
        pltpu.VMEM((CHUNK,), jnp.float32),
        pltpu.VMEM((ROWS_PER_TILE,), jnp.float32),
        pltpu.VMEM_SHARED((N_PAD,), jnp.float32),
    ],
)


def _sc_agg_body(srcA_hbm, dstA_hbm, srcB_hbm, dstB_hbm, g_hbm, acc_out,
                 isrc_v, idst_v, rows0, rows1, acc_sh, gsem0, gsem1):
    rows = [rows0, rows1]
    gsem = [gsem0, gsem1]
    c = lax.axis_index("c")
    s = lax.axis_index("s")
    zeros16 = jnp.zeros((16,), jnp.float32)

    def zbody(j, carry):
        for k in range(D // 16):
            rows0[j, pl.ds(k * 16, 16)] = zeros16
        return carry

    lax.fori_loop(0, CHUNK, zbody, 0)
    for k in range(ROWS_PER_TILE // CHUNK):
        pltpu.sync_copy(
            rows0, acc_sh.at[pl.ds(s * ROWS_PER_TILE + k * CHUNK, CHUNK)])
    plsc.subcore_barrier()

    def run(src_ref, dst_ref, nparts):
        for part in range(nparts):
            pltpu.sync_copy(src_ref.at[s, pl.ds(part * PART, PART)], isrc_v)
            pltpu.sync_copy(dst_ref.at[s, pl.ds(part * PART, PART)], idst_v)
            pltpu.async_copy(g_hbm.at[isrc_v.at[0]], rows0, gsem[0])

            def body(jj, carry):
                for t in range(2):
                    j = jj * 2 + t
                    b, bn = rows[t], rows[1 - t]
                    sb, sbn = gsem[t], gsem[1 - t]
                    pltpu.make_async_copy(g_hbm.at[isrc_v.at[j]], b,
                                          sb).wait()

                    def prefetch(j=j, bn=bn, sbn=sbn):
                        pltpu.async_copy(g_hbm.at[isrc_v.at[j + 1]], bn, sbn)

                    pl.when(j < PART - 1)(prefetch)
                    pltpu.sync_copy(b, acc_sh.at[idst_v.at[j]], add=True)
                return carry

            lax.fori_loop(0, PART // 2, body, 0)

    pl.when(c == 0)(lambda: run(srcA_hbm, dstA_hbm, CHUNKS_C0 // PART))
    pl.when(c == 1)(lambda: run(srcB_hbm, dstB_hbm, CHUNKS_C1 // PART))
    plsc.subcore_barrier()
    pltpu.sync_copy(acc_sh.at[pl.ds(s * ROWS_PER_TILE, ROWS_PER_TILE)],
                    acc_out.at[c, pl.ds(s * ROWS_PER_TILE, ROWS_PER_TILE)])


_sc_agg = pl.kernel(
    _sc_agg_body,
    out_type=jax.ShapeDtypeStruct((NC, N_PAD, D), jnp.float32),
    mesh=_sc_mesh(),
    scratch_types=[
        pltpu.VMEM((PART, CHUNK), jnp.int32),
        pltpu.VMEM((PART, CHUNK), jnp.int32),
        pltpu.VMEM((CHUNK, D), jnp.float32),
        pltpu.VMEM((CHUNK, D), jnp.float32),
        pltpu.VMEM_SHARED((N_PAD, D), jnp.float32),
        pltpu.SemaphoreType.DMA,
        pltpu.SemaphoreType.DMA,
    ],
)


def _tc_transform_body(x_ref, w_ref, deg0_ref, deg1_ref, h_ref, g_ref):
    h = jnp.dot(x_ref[...], w_ref[...], preferred_element_type=jnp.float32)
    deg = deg0_ref[...] + deg1_ref[...] + 1.0
    dis = lax.rsqrt(deg)
    h_ref[...] = h
    g_ref[...] = h * dis


def _tc_transform(x, W, deg0, deg1):
    R = 1000
    grid = (N // R,)
    return pl.pallas_call(
        _tc_transform_body,
        grid=grid,
        in_specs=[
            pl.BlockSpec((R, D), lambda i: (i, 0)),
            pl.BlockSpec((D, D), lambda i: (0, 0)),
            pl.BlockSpec((R, 1), lambda i: (i, 0)),
            pl.BlockSpec((R, 1), lambda i: (i, 0)),
        ],
        out_specs=[
            pl.BlockSpec((R, D), lambda i: (i, 0)),
            pl.BlockSpec((R, D), lambda i: (i, 0)),
        ],
        out_shape=[
            jax.ShapeDtypeStruct((N, D), jnp.float32),
            jax.ShapeDtypeStruct((N, D), jnp.float32),
        ],
    )(x, W, deg0, deg1)


def _tc_final_body(h_ref, deg0_ref, deg1_ref, acc0_ref, acc1_ref, b_ref,
                   gamma_ref, beta_ref, a_ref, o_ref):
    deg = deg0_ref[...] + deg1_ref[...] + 1.0
    dis = lax.rsqrt(deg)
    acc = acc0_ref[...] + acc1_ref[...]
    out = acc * dis + h_ref[...] * (1.0 / deg) + b_ref[...]
    mu = jnp.mean(out, axis=1, keepdims=True)
    cen = out - mu
    var = jnp.mean(cen * cen, axis=1, keepdims=True)
    y = cen * lax.rsqrt(var + 1e-5) * gamma_ref[...] + beta_ref[...]
    o_ref[...] = jnp.where(y >= 0.0, y, a_ref[...] * y)


def _tc_final(h, deg0, deg1, acc0, acc1, b, gamma, beta, a):
    R = 1000
    grid = (N // R,)
    full = lambda i: (0, 0)
    return pl.pallas_call(
        _tc_final_body,
        grid=grid,
        in_specs=[
            pl.BlockSpec((R, D), lambda i: (i, 0)),
            pl.BlockSpec((R, 1), lambda i: (i, 0)),
            pl.BlockSpec((R, 1), lambda i: (i, 0)),
            pl.BlockSpec((R, D), lambda i: (i, 0)),
            pl.BlockSpec((R, D), lambda i: (i, 0)),
            pl.BlockSpec((1, D), full),
            pl.BlockSpec((1, D), full),
            pl.BlockSpec((1, D), full),
            pl.BlockSpec((1, 1), full),
        ],
        out_specs=pl.BlockSpec((R, D), lambda i: (i, 0)),
        out_shape=jax.ShapeDtypeStruct((N, D), jnp.float32),
    )(h, deg0, deg1, acc0, acc1, b, gamma, beta, a)


def kernel(x, edge_index, batch, W, b, gamma, beta, a):
    del batch
    src = edge_index[0].astype(jnp.int32)
    dst = edge_index[1].astype(jnp.int32)
    n_dummy = E_PAD - E
    src_p = jnp.concatenate([src, jnp.zeros((n_dummy,), jnp.int32)])
    dst_p = jnp.concatenate(
        [dst, N + (jnp.arange(n_dummy, dtype=jnp.int32) % (N_PAD - N))])
    dst_r = dst_p.reshape(NW, E_PER_TILE_CHUNKS, CHUNK)
    e_a = NS * CHUNKS_C0 * CHUNK
    src_a = src_p[:e_a].reshape(NS, CHUNKS_C0, CHUNK)
    dst_a = dst_p[:e_a].reshape(NS, CHUNKS_C0, CHUNK)
    src_b = src_p[e_a:].reshape(NS, CHUNKS_C1, CHUNK)
    dst_b = dst_p[e_a:].reshape(NS, CHUNKS_C1, CHUNK)

    degp = _sc_hist(dst_r)
    deg0 = degp[0, :N].reshape(N, 1)
    deg1 = degp[1, :N].reshape(N, 1)

    h, g = _tc_transform(x, W, deg0, deg1)

    acc = _sc_agg(src_a, dst_a, src_b, dst_b, g)
    acc0 = acc[0, :N]
    acc1 = acc[1, :N]

    b2 = b.reshape(1, D)
    gamma2 = gamma.reshape(1, D)
    beta2 = beta.reshape(1, D)
    a2 = a.reshape(1, 1)
    return _tc_final(h, deg0, deg1, acc0, acc1, b2, gamma2, beta2, a2)
